# Initial kernel scaffold; baseline (speedup 1.0000x reference)
import jax
import jax.numpy as jnp
from jax.experimental import pallas as pl


def kernel(x, edge_index, edge_attr, batch, node_emb, edge_W, edge_b, W1, b1, W2, b2, gamma, beta, pred_W, pred_b):
    raise NotImplementedError("write your pallas kernel here")


# speedup vs baseline: 3.3030x; 3.3030x over previous
import functools

import jax
import jax.numpy as jnp
from jax import lax
from jax.experimental import pallas as pl
from jax.experimental.pallas import tpu as pltpu
from jax.experimental.pallas import tpu_sc as plsc

N_NODES = 10000
N_EDGES = 320000
DIM = 128
NUM_GRAPHS = 128
NUM_CLASSES = 10
DE_PAD = 8

NUM_CORES = 2
NUM_SUBCORES = 16
NUM_WORKERS = NUM_CORES * NUM_SUBCORES
EDGES_PER_WORKER = N_EDGES // NUM_WORKERS
CHUNK = 80
NUM_CHUNKS = EDGES_PER_WORKER // CHUNK
N_PAD = 10240
ROWS_PER_SUB = N_PAD // NUM_SUBCORES
ZROWS = 128
LANES = 16

@functools.cache
def _sc_mesh():
    return plsc.VectorSubcoreMesh(
        core_axis_name="c", subcore_axis_name="s",
        num_cores=NUM_CORES, num_subcores=NUM_SUBCORES)


def _edge_encode_body(ea_ref, w0_ref, b0_ref, h0_ref, w1_ref, b1_ref,
                      msg0_ref, ee1_ref):
    a = ea_ref[...]
    e0 = jnp.dot(a, w0_ref[...], preferred_element_type=jnp.float32) + b0_ref[...]
    msg0_ref[...] = jnp.maximum(e0 + h0_ref[...], 0.0)
    ee1_ref[...] = jnp.dot(a, w1_ref[...], preferred_element_type=jnp.float32) + b1_ref[...]


_EB = 4000


def _edge_encode(ea, w0, b0, h0, w1, b1):
    grid = (N_EDGES // _EB,)
    return pl.pallas_call(
        _edge_encode_body,
        grid=grid,
        in_specs=[
            pl.BlockSpec((_EB, DE_PAD), lambda i: (i, 0)),
            pl.BlockSpec((DE_PAD, DIM), lambda i: (0, 0)),
            pl.BlockSpec((1, DIM), lambda i: (0, 0)),
            pl.BlockSpec((1, DIM), lambda i: (0, 0)),
            pl.BlockSpec((DE_PAD, DIM), lambda i: (0, 0)),
            pl.BlockSpec((1, DIM), lambda i: (0, 0)),
        ],
        out_specs=[
            pl.BlockSpec((_EB, DIM), lambda i: (i, 0)),
            pl.BlockSpec((_EB, DIM), lambda i: (i, 0)),
        ],
        out_shape=[
            jax.ShapeDtypeStruct((N_EDGES, DIM), jnp.float32),
            jax.ShapeDtypeStruct((N_EDGES, DIM), jnp.float32),
        ],
    )(ea, w0, b0, h0, w1, b1)


def _gin0_body(h0_ref, agg_ref, w1_ref, b1_ref, w2_ref, b2_ref, g_ref, be_ref,
               out_ref):
    agg = agg_ref[0, :N_NODES, :] + agg_ref[1, :N_NODES, :]
    z = h0_ref[...] + agg
    t = jnp.maximum(jnp.dot(z, w1_ref[...], preferred_element_type=jnp.float32)
                    + b1_ref[...], 0.0)
    t = jnp.dot(t, w2_ref[...], preferred_element_type=jnp.float32) + b2_ref[...]
    mu = jnp.mean(t, axis=0, keepdims=True)
    var = jnp.mean((t - mu) ** 2, axis=0, keepdims=True)
    t = (t - mu) * lax.rsqrt(var + 1e-5) * g_ref[...] + be_ref[...]
    out_ref[...] = jnp.maximum(t, 0.0)


def _gin0(h0, agg, w1, b1, w2, b2, g, be):
    return pl.pallas_call(
        _gin0_body,
        out_shape=jax.ShapeDtypeStruct((N_NODES, DIM), jnp.float32),
    )(h0, agg, w1, b1, w2, b2, g, be)


def _final_body(h_ref, agg_ref, w1_ref, b1_ref, w2_ref, b2_ref, g_ref, be_ref,
                batch_ref, pw_ref, pb_ref, out_ref):
    agg = agg_ref[0, :N_NODES, :] + agg_ref[1, :N_NODES, :]
    z = h_ref[...] + agg
    t = jnp.maximum(jnp.dot(z, w1_ref[...], preferred_element_type=jnp.float32)
                    + b1_ref[...], 0.0)
    t = jnp.dot(t, w2_ref[...], preferred_element_type=jnp.float32) + b2_ref[...]
    mu = jnp.mean(t, axis=0, keepdims=True)
    var = jnp.mean((t - mu) ** 2, axis=0, keepdims=True)
    t = (t - mu) * lax.rsqrt(var + 1e-5) * g_ref[...] + be_ref[...]
    onehot = (batch_ref[...] ==
              lax.broadcasted_iota(jnp.int32, (N_NODES, NUM_GRAPHS), 1)
              ).astype(jnp.float32)
    pooled = lax.dot_general(onehot, t, (((0,), (0,)), ((), ())),
                             preferred_element_type=jnp.float32)
    counts = jnp.sum(onehot, axis=0)
    pooled = pooled / jnp.maximum(counts, 1.0)[:, None]
    out_ref[...] = (jnp.dot(pooled, pw_ref[...],
                            preferred_element_type=jnp.float32) + pb_ref[...])


def _final(h, agg, w1, b1, w2, b2, g, be, batch2d, pw, pb):
    return pl.pallas_call(
        _final_body,
        out_shape=jax.ShapeDtypeStruct((NUM_GRAPHS, NUM_CLASSES), jnp.float32),
    )(h, agg, w1, b1, w2, b2, g, be, batch2d, pw, pb)


def _zero_vmem(ref, rows):
    z = jnp.zeros((LANES,), jnp.float32)

    def body(r, carry):
        for j in range(DIM // LANES):
            ref[r, pl.ds(j * LANES, LANES)] = z
        return carry

    lax.fori_loop(0, rows, body, 0)


def _zero_agg(zbuf, agg_sh, s):
    _zero_vmem(zbuf, ZROWS)
    base = s * ROWS_PER_SUB
    for k in range(ROWS_PER_SUB // ZROWS):
        pltpu.sync_copy(zbuf, agg_sh.at[pl.ds(base + k * ZROWS, ZROWS)])


def _sc_scatter0_body(msg_hbm, dst_hbm, out_hbm, zbuf, msg_v, dst_v, agg_sh):
    c = lax.axis_index("c")
    s = lax.axis_index("s")
    wid = c * NUM_SUBCORES + s
    _zero_agg(zbuf, agg_sh, s)
    plsc.subcore_barrier()
    ebase = wid * EDGES_PER_WORKER

    def body(i, carry):
        b = ebase + i * CHUNK
        pltpu.sync_copy(dst_hbm.at[pl.ds(b, CHUNK)], dst_v)
        pltpu.sync_copy(msg_hbm.at[pl.ds(b, CHUNK)], msg_v)
        pltpu.sync_copy(msg_v, agg_sh.at[dst_v], add=True)
        return carry

    lax.fori_loop(0, NUM_CHUNKS, body, 0)
    plsc.subcore_barrier()
    r0 = s * ROWS_PER_SUB
    pltpu.sync_copy(agg_sh.at[pl.ds(r0, ROWS_PER_SUB)],
                    out_hbm.at[c, pl.ds(r0, ROWS_PER_SUB)])


def _sc_gather1_body(h_hbm, ee_hbm, src_hbm, dst_hbm, out_hbm,
                     zbuf, rows_v, ee_v, src_v, dst_v, agg_sh, sem):
    c = lax.axis_index("c")
    s = lax.axis_index("s")
    wid = c * NUM_SUBCORES + s
    _zero_agg(zbuf, agg_sh, s)
    plsc.subcore_barrier()
    ebase = wid * EDGES_PER_WORKER

    def body(i, carry):
        b = ebase + i * CHUNK
        pltpu.sync_copy(src_hbm.at[pl.ds(b, CHUNK)], src_v)
        pltpu.sync_copy(dst_hbm.at[pl.ds(b, CHUNK)], dst_v)
        pltpu.async_copy(h_hbm.at[src_v], rows_v, sem).wait()
        pltpu.sync_copy(ee_hbm.at[pl.ds(b, CHUNK)], ee_v)

        def rbody(r, rc):
            for j in range(DIM // LANES):
                sl = pl.ds(j * LANES, LANES)
                rows_v[r, sl] = jnp.maximum(rows_v[r, sl] + ee_v[r, sl], 0.0)
            return rc

        lax.fori_loop(0, CHUNK, rbody, 0)
        pltpu.sync_copy(rows_v, agg_sh.at[dst_v], add=True)
        return carry

    lax.fori_loop(0, NUM_CHUNKS, body, 0)
    plsc.subcore_barrier()
    r0 = s * ROWS_PER_SUB
    pltpu.sync_copy(agg_sh.at[pl.ds(r0, ROWS_PER_SUB)],
                    out_hbm.at[c, pl.ds(r0, ROWS_PER_SUB)])


@functools.cache
def _sc_scatter0():
    return pl.kernel(
        _sc_scatter0_body,
        out_type=jax.ShapeDtypeStruct((NUM_CORES, N_PAD, DIM), jnp.float32),
        mesh=_sc_mesh(),
        scratch_types=[
            pltpu.VMEM((ZROWS, DIM), jnp.float32),
            pltpu.VMEM((CHUNK, DIM), jnp.float32),
            pltpu.VMEM((CHUNK,), jnp.int32),
            pltpu.VMEM_SHARED((N_PAD, DIM), jnp.float32),
        ],
    )


@functools.cache
def _sc_gather1():
    return pl.kernel(
        _sc_gather1_body,
        out_type=jax.ShapeDtypeStruct((NUM_CORES, N_PAD, DIM), jnp.float32),
        mesh=_sc_mesh(),
        scratch_types=[
            pltpu.VMEM((ZROWS, DIM), jnp.float32),
            pltpu.VMEM((CHUNK, DIM), jnp.float32),
            pltpu.VMEM((CHUNK, DIM), jnp.float32),
            pltpu.VMEM((CHUNK,), jnp.int32),
            pltpu.VMEM((CHUNK,), jnp.int32),
            pltpu.VMEM_SHARED((N_PAD, DIM), jnp.float32),
            pltpu.SemaphoreType.DMA,
        ],
    )


def kernel(x, edge_index, edge_attr, batch, node_emb, edge_W, edge_b,
           W1, b1, W2, b2, gamma, beta, pred_W, pred_b):
    src = edge_index[0]
    dst = edge_index[1]
    h0 = node_emb[0:1]
    ea = jnp.concatenate(
        [edge_attr, jnp.zeros((N_EDGES, DE_PAD - edge_attr.shape[1]),
                              jnp.float32)], axis=1)
    w0 = jnp.concatenate([edge_W[0], jnp.zeros((1, DIM), jnp.float32)], axis=0)
    w1e = jnp.concatenate([edge_W[1], jnp.zeros((1, DIM), jnp.float32)], axis=0)

    msg0, ee1 = _edge_encode(ea, w0, edge_b[0:1], h0, w1e, edge_b[1:2])
    agg0 = _sc_scatter0()(msg0, dst)
    h1 = _gin0(h0, agg0, W1[0], b1[0:1], W2[0], b2[0:1], gamma[0:1], beta[0:1])
    agg1 = _sc_gather1()(h1, ee1, src, dst)
    return _final(h1, agg1, W1[1], b1[1:2], W2[1], b2[1:2],
                  gamma[1:2], beta[1:2], batch.reshape(N_NODES, 1),
                  pred_W, pred_b)


# speedup vs baseline: 5.1886x; 1.5709x over previous
import functools

import jax
import jax.numpy as jnp
from jax import lax
from jax.experimental import pallas as pl
from jax.experimental.pallas import tpu as pltpu
from jax.experimental.pallas import tpu_sc as plsc

N_NODES = 10000
N_EDGES = 320000
DIM = 128
NUM_GRAPHS = 128
NUM_CLASSES = 10
DE_PAD = 8

NUM_CORES = 2
NUM_SUBCORES = 16
NUM_WORKERS = NUM_CORES * NUM_SUBCORES
EDGES_PER_WORKER = N_EDGES // NUM_WORKERS
CHUNK = 80
NUM_CHUNKS = EDGES_PER_WORKER // CHUNK
N_PAD = 10240
ROWS_PER_SUB = N_PAD // NUM_SUBCORES
ZROWS = 128
LANES = 16

@functools.cache
def _sc_mesh():
    return plsc.VectorSubcoreMesh(
        core_axis_name="c", subcore_axis_name="s",
        num_cores=NUM_CORES, num_subcores=NUM_SUBCORES)


def _edge_encode_body(ea_ref, w0_ref, b0_ref, h0_ref, w1_ref, b1_ref,
                      msg0_ref, ee1_ref):
    a = ea_ref[...]
    e0 = jnp.dot(a, w0_ref[...], preferred_element_type=jnp.float32) + b0_ref[...]
    msg0_ref[...] = jnp.maximum(e0 + h0_ref[...], 0.0)
    ee1_ref[...] = jnp.dot(a, w1_ref[...], preferred_element_type=jnp.float32) + b1_ref[...]


_EB = 4000


def _edge_encode(ea, w0, b0, h0, w1, b1):
    grid = (N_EDGES // _EB,)
    return pl.pallas_call(
        _edge_encode_body,
        grid=grid,
        in_specs=[
            pl.BlockSpec((_EB, DE_PAD), lambda i: (i, 0)),
            pl.BlockSpec((DE_PAD, DIM), lambda i: (0, 0)),
            pl.BlockSpec((1, DIM), lambda i: (0, 0)),
            pl.BlockSpec((1, DIM), lambda i: (0, 0)),
            pl.BlockSpec((DE_PAD, DIM), lambda i: (0, 0)),
            pl.BlockSpec((1, DIM), lambda i: (0, 0)),
        ],
        out_specs=[
            pl.BlockSpec((_EB, DIM), lambda i: (i, 0)),
            pl.BlockSpec((_EB, DIM), lambda i: (i, 0)),
        ],
        out_shape=[
            jax.ShapeDtypeStruct((N_EDGES, DIM), jnp.float32),
            jax.ShapeDtypeStruct((N_EDGES, DIM), jnp.float32),
        ],
    )(ea, w0, b0, h0, w1, b1)


def _gin0_body(h0_ref, agg_ref, w1_ref, b1_ref, w2_ref, b2_ref, g_ref, be_ref,
               out_ref):
    agg = agg_ref[0, :N_NODES, :] + agg_ref[1, :N_NODES, :]
    z = h0_ref[...] + agg
    t = jnp.maximum(jnp.dot(z, w1_ref[...], preferred_element_type=jnp.float32)
                    + b1_ref[...], 0.0)
    t = jnp.dot(t, w2_ref[...], preferred_element_type=jnp.float32) + b2_ref[...]
    mu = jnp.mean(t, axis=0, keepdims=True)
    var = jnp.mean((t - mu) ** 2, axis=0, keepdims=True)
    t = (t - mu) * lax.rsqrt(var + 1e-5) * g_ref[...] + be_ref[...]
    out_ref[...] = jnp.maximum(t, 0.0)


def _gin0(h0, agg, w1, b1, w2, b2, g, be):
    return pl.pallas_call(
        _gin0_body,
        out_shape=jax.ShapeDtypeStruct((N_NODES, DIM), jnp.float32),
    )(h0, agg, w1, b1, w2, b2, g, be)


def _final_body(h_ref, agg_ref, w1_ref, b1_ref, w2_ref, b2_ref, g_ref, be_ref,
                batch_ref, pw_ref, pb_ref, out_ref):
    agg = agg_ref[0, :N_NODES, :] + agg_ref[1, :N_NODES, :]
    z = h_ref[...] + agg
    t = jnp.maximum(jnp.dot(z, w1_ref[...], preferred_element_type=jnp.float32)
                    + b1_ref[...], 0.0)
    t = jnp.dot(t, w2_ref[...], preferred_element_type=jnp.float32) + b2_ref[...]
    mu = jnp.mean(t, axis=0, keepdims=True)
    var = jnp.mean((t - mu) ** 2, axis=0, keepdims=True)
    t = (t - mu) * lax.rsqrt(var + 1e-5) * g_ref[...] + be_ref[...]
    onehot = (batch_ref[...] ==
              lax.broadcasted_iota(jnp.int32, (N_NODES, NUM_GRAPHS), 1)
              ).astype(jnp.float32)
    pooled = lax.dot_general(onehot, t, (((0,), (0,)), ((), ())),
                             preferred_element_type=jnp.float32)
    counts = jnp.sum(onehot, axis=0)
    pooled = pooled / jnp.maximum(counts, 1.0)[:, None]
    out_ref[...] = (jnp.dot(pooled, pw_ref[...],
                            preferred_element_type=jnp.float32) + pb_ref[...])


def _final(h, agg, w1, b1, w2, b2, g, be, batch2d, pw, pb):
    return pl.pallas_call(
        _final_body,
        out_shape=jax.ShapeDtypeStruct((NUM_GRAPHS, NUM_CLASSES), jnp.float32),
    )(h, agg, w1, b1, w2, b2, g, be, batch2d, pw, pb)


def _zero_vmem(ref, rows):
    z = jnp.zeros((LANES,), jnp.float32)

    def body(r, carry):
        for j in range(DIM // LANES):
            ref[r, pl.ds(j * LANES, LANES)] = z
        return carry

    lax.fori_loop(0, rows, body, 0)


def _zero_agg(zbuf, agg_sh, s):
    _zero_vmem(zbuf, CHUNK)
    base = s * ROWS_PER_SUB
    for k in range(ROWS_PER_SUB // CHUNK):
        pltpu.sync_copy(zbuf, agg_sh.at[pl.ds(base + k * CHUNK, CHUNK)])


def _sc_scatter0_body(msg_hbm, dst3_hbm, out_hbm,
                      msg_v0, msg_v1, dst_v0, dst_v1, agg_sh,
                      sem0, sem1, isem0, isem1):
    c = lax.axis_index("c")
    s = lax.axis_index("s")
    wid = c * NUM_SUBCORES + s
    ebase = wid * EDGES_PER_WORKER
    _zero_agg(msg_v0, agg_sh, s)
    plsc.subcore_barrier()

    msgs = (msg_v0, msg_v1)
    dsts = (dst_v0, dst_v1)
    sems = (sem0, sem1)
    isems = (isem0, isem1)

    def data_start(i, b):
        pltpu.async_copy(msg_hbm.at[pl.ds(ebase + i * CHUNK, CHUNK)],
                         msgs[b], sems[b])

    def data_wait(i, b):
        pltpu.make_async_copy(msg_hbm.at[pl.ds(ebase + i * CHUNK, CHUNK)],
                              msgs[b], sems[b]).wait()

    def idx_start(j, b):
        pltpu.async_copy(dst3_hbm.at[wid, j], dsts[b], isems[b])

    def idx_wait(j, b):
        pltpu.make_async_copy(dst3_hbm.at[wid, j], dsts[b], isems[b]).wait()

    pltpu.sync_copy(dst3_hbm.at[wid, 0], dst_v0)
    data_start(0, 0)
    idx_start(1, 1)

    def body(i, carry):
        def process(b):
            data_wait(i, b)

            @pl.when(i + 1 < NUM_CHUNKS)
            def _():
                idx_wait(i + 1, 1 - b)
                data_start(i + 1, 1 - b)

            pltpu.sync_copy(msgs[b], agg_sh.at[dsts[b]], add=True)

            @pl.when(i + 2 < NUM_CHUNKS)
            def _():
                idx_start(i + 2, b)

        lax.cond(i % 2 == 0, lambda: process(0), lambda: process(1))
        return carry

    lax.fori_loop(0, NUM_CHUNKS, body, 0)
    plsc.subcore_barrier()
    r0 = s * ROWS_PER_SUB
    pltpu.sync_copy(agg_sh.at[pl.ds(r0, ROWS_PER_SUB)],
                    out_hbm.at[c, pl.ds(r0, ROWS_PER_SUB)])


def _sc_gather1_body(h_hbm, ee_hbm, src3_hbm, dst3_hbm, out_hbm,
                     rows_v0, rows_v1, ee_v0, ee_v1,
                     src_v0, src_v1, dst_v0, dst_v1, agg_sh,
                     gsem0, gsem1, esem0, esem1, isem0, isem1):
    c = lax.axis_index("c")
    s = lax.axis_index("s")
    wid = c * NUM_SUBCORES + s
    ebase = wid * EDGES_PER_WORKER
    _zero_agg(rows_v0, agg_sh, s)
    plsc.subcore_barrier()

    rows = (rows_v0, rows_v1)
    ees = (ee_v0, ee_v1)
    srcs = (src_v0, src_v1)
    dsts = (dst_v0, dst_v1)
    gsems = (gsem0, gsem1)
    esems = (esem0, esem1)
    isems = (isem0, isem1)

    def data_start(i, b):
        pltpu.async_copy(ee_hbm.at[pl.ds(ebase + i * CHUNK, CHUNK)],
                         ees[b], esems[b])
        pltpu.async_copy(h_hbm.at[srcs[b]], rows[b], gsems[b])

    def data_wait(i, b):
        pltpu.make_async_copy(ee_hbm.at[pl.ds(ebase + i * CHUNK, CHUNK)],
                              ees[b], esems[b]).wait()
        pltpu.make_async_copy(h_hbm.at[srcs[b]], rows[b], gsems[b]).wait()

    def idx_start(j, b):
        pltpu.async_copy(src3_hbm.at[wid, j], srcs[b], isems[b])
        pltpu.async_copy(dst3_hbm.at[wid, j], dsts[b], isems[b])

    def idx_wait(j, b):
        pltpu.make_async_copy(src3_hbm.at[wid, j], srcs[b], isems[b]).wait()
        pltpu.make_async_copy(dst3_hbm.at[wid, j], dsts[b], isems[b]).wait()

    pltpu.sync_copy(src3_hbm.at[wid, 0], src_v0)
    pltpu.sync_copy(dst3_hbm.at[wid, 0], dst_v0)
    data_start(0, 0)
    idx_start(1, 1)

    def body(i, carry):
        def process(b):
            data_wait(i, b)

            @pl.when(i + 1 < NUM_CHUNKS)
            def _():
                idx_wait(i + 1, 1 - b)
                data_start(i + 1, 1 - b)

            rbuf = rows[b]
            ebuf = ees[b]

            def rbody(r, rc):
                for j in range(DIM // LANES):
                    sl = pl.ds(j * LANES, LANES)
                    rbuf[r, sl] = jnp.maximum(rbuf[r, sl] + ebuf[r, sl], 0.0)
                return rc

            lax.fori_loop(0, CHUNK, rbody, 0)
            pltpu.sync_copy(rbuf, agg_sh.at[dsts[b]], add=True)

            @pl.when(i + 2 < NUM_CHUNKS)
            def _():
                idx_start(i + 2, b)

        lax.cond(i % 2 == 0, lambda: process(0), lambda: process(1))
        return carry

    lax.fori_loop(0, NUM_CHUNKS, body, 0)
    plsc.subcore_barrier()
    r0 = s * ROWS_PER_SUB
    pltpu.sync_copy(agg_sh.at[pl.ds(r0, ROWS_PER_SUB)],
                    out_hbm.at[c, pl.ds(r0, ROWS_PER_SUB)])


@functools.cache
def _sc_scatter0():
    return pl.kernel(
        _sc_scatter0_body,
        out_type=jax.ShapeDtypeStruct((NUM_CORES, N_PAD, DIM), jnp.float32),
        mesh=_sc_mesh(),
        scratch_types=[
            pltpu.VMEM((CHUNK, DIM), jnp.float32),
            pltpu.VMEM((CHUNK, DIM), jnp.float32),
            pltpu.VMEM((CHUNK,), jnp.int32),
            pltpu.VMEM((CHUNK,), jnp.int32),
            pltpu.VMEM_SHARED((N_PAD, DIM), jnp.float32),
            pltpu.SemaphoreType.DMA,
            pltpu.SemaphoreType.DMA,
            pltpu.SemaphoreType.DMA,
            pltpu.SemaphoreType.DMA,
        ],
    )


@functools.cache
def _sc_gather1():
    return pl.kernel(
        _sc_gather1_body,
        out_type=jax.ShapeDtypeStruct((NUM_CORES, N_PAD, DIM), jnp.float32),
        mesh=_sc_mesh(),
        scratch_types=[
            pltpu.VMEM((CHUNK, DIM), jnp.float32),
            pltpu.VMEM((CHUNK, DIM), jnp.float32),
            pltpu.VMEM((CHUNK, DIM), jnp.float32),
            pltpu.VMEM((CHUNK, DIM), jnp.float32),
            pltpu.VMEM((CHUNK,), jnp.int32),
            pltpu.VMEM((CHUNK,), jnp.int32),
            pltpu.VMEM((CHUNK,), jnp.int32),
            pltpu.VMEM((CHUNK,), jnp.int32),
            pltpu.VMEM_SHARED((N_PAD, DIM), jnp.float32),
            pltpu.SemaphoreType.DMA,
            pltpu.SemaphoreType.DMA,
            pltpu.SemaphoreType.DMA,
            pltpu.SemaphoreType.DMA,
            pltpu.SemaphoreType.DMA,
            pltpu.SemaphoreType.DMA,
        ],
    )


def kernel(x, edge_index, edge_attr, batch, node_emb, edge_W, edge_b,
           W1, b1, W2, b2, gamma, beta, pred_W, pred_b):
    src = edge_index[0]
    dst = edge_index[1]
    h0 = node_emb[0:1]
    ea = jnp.concatenate(
        [edge_attr, jnp.zeros((N_EDGES, DE_PAD - edge_attr.shape[1]),
                              jnp.float32)], axis=1)
    w0 = jnp.concatenate([edge_W[0], jnp.zeros((1, DIM), jnp.float32)], axis=0)
    w1e = jnp.concatenate([edge_W[1], jnp.zeros((1, DIM), jnp.float32)], axis=0)

    src3 = src.reshape(NUM_WORKERS, NUM_CHUNKS, CHUNK)
    dst3 = dst.reshape(NUM_WORKERS, NUM_CHUNKS, CHUNK)
    msg0, ee1 = _edge_encode(ea, w0, edge_b[0:1], h0, w1e, edge_b[1:2])
    agg0 = _sc_scatter0()(msg0, dst3)
    h1 = _gin0(h0, agg0, W1[0], b1[0:1], W2[0], b2[0:1], gamma[0:1], beta[0:1])
    agg1 = _sc_gather1()(h1, ee1, src3, dst3)
    return _final(h1, agg1, W1[1], b1[1:2], W2[1], b2[1:2],
                  gamma[1:2], beta[1:2], batch.reshape(N_NODES, 1),
                  pred_W, pred_b)
